# baseline (speedup 1.0000x reference)
import functools
import jax
import jax.numpy as jnp
from jax import lax
from jax.experimental import pallas as pl
from jax.experimental.pallas import tpu as pltpu
from jax.experimental.pallas import tpu_sc as plsc

B, S, D = 4, 8192, 1024
NW = 32
CH = (B * S * D) // NW
WPB = NW // B
T = 16384


def _sc_body(x_hbm, pos_hbm, out_hbm, xv, pv):
    w = lax.axis_index("s") * 2 + lax.axis_index("c")
    base = w * CH
    pbase = (w % WPB) * CH

    def step(i, carry):
        off = i * T
        pltpu.sync_copy(x_hbm.at[pl.ds(base + off, T)], xv)
        pltpu.sync_copy(pos_hbm.at[pl.ds(pbase + off, T)], pv)

        def inner(j, c):
            for k in range(4):
                sl = pl.ds(j * 64 + k * 16, 16)
                xv[sl] = xv[sl] + pv[sl]
            return c

        lax.fori_loop(0, T // 64, inner, 0)
        pltpu.sync_copy(xv, out_hbm.at[pl.ds(base + off, T)])
        return carry

    lax.fori_loop(0, CH // T, step, 0)


def kernel(x, pos_table):
    xf = x.reshape(-1)
    pf = pos_table.reshape(-1)
    mesh = plsc.VectorSubcoreMesh(core_axis_name="c", subcore_axis_name="s")
    run = functools.partial(
        pl.kernel,
        mesh=mesh,
        out_type=jax.ShapeDtypeStruct((B * S * D,), jnp.float32),
        scratch_types=[
            pltpu.VMEM((T,), jnp.float32),
            pltpu.VMEM((T,), jnp.float32),
        ],
    )(_sc_body)
    return run(xf, pf).reshape(B, S, D)


# speedup vs baseline: 1.4678x; 1.4678x over previous
import functools
import jax
import jax.numpy as jnp
from jax import lax
from jax.experimental import pallas as pl
from jax.experimental.pallas import tpu as pltpu
from jax.experimental.pallas import tpu_sc as plsc

B, S, D = 4, 8192, 1024
NW = 32
ROWS_W = S // NW
TR = 16
T = TR * D
NT = ROWS_W // TR
PB = S * D
NIT = NT * B


def _sc_body(x_hbm, pos_hbm, out_hbm,
             xv0, xv1, ov0, ov1, pv0, pv1,
             sin0, sin1, sout0, sout1, spos0, spos1):
    xv = (xv0, xv1)
    ov = (ov0, ov1)
    pv = (pv0, pv1)
    sin = (sin0, sin1)
    sout = (sout0, sout1)
    spos = (spos0, spos1)

    w = lax.axis_index("s") * 2 + lax.axis_index("c")
    prow = w * (ROWS_W * D)

    def x_off(g, b):
        return b * PB + prow + g * T

    def p_off(g):
        return prow + g * T

    pltpu.make_async_copy(pos_hbm.at[pl.ds(p_off(0), T)], pv0, spos0).start()
    pltpu.make_async_copy(pos_hbm.at[pl.ds(p_off(1), T)], pv1, spos1).start()
    pltpu.make_async_copy(x_hbm.at[pl.ds(x_off(0, 0), T)], xv0, sin0).start()
    pltpu.make_async_copy(x_hbm.at[pl.ds(x_off(0, 1), T)], xv1, sin1).start()

    def group2(g2, carry):
        for gg in range(2):
            g = g2 * 2 + gg
            q = gg
            for b in range(4):
                s = b % 2
                it = g * 4 + b
                pltpu.make_async_copy(
                    x_hbm.at[pl.ds(x_off(g, b), T)], xv[s], sin[s]).wait()
                if b == 0:
                    pltpu.make_async_copy(
                        pos_hbm.at[pl.ds(p_off(g), T)], pv[q], spos[q]).wait()

                if gg * 4 + b >= 2:
                    pltpu.make_async_copy(
                        ov[s], out_hbm.at[pl.ds(x_off(g, b), T)],
                        sout[s]).wait()
                else:
                    @pl.when(g2 >= 1)
                    def _drain():
                        pltpu.make_async_copy(
                            ov[s], out_hbm.at[pl.ds(x_off(g, b), T)],
                            sout[s]).wait()

                def inner(j, c):
                    for k in range(4):
                        sl = pl.ds(j * 64 + k * 16, 16)
                        ov[s][sl] = xv[s][sl] + pv[q][sl]
                    return c

                lax.fori_loop(0, T // 64, inner, 0)

                pltpu.make_async_copy(
                    ov[s], out_hbm.at[pl.ds(x_off(g, b), T)], sout[s]).start()

                nb = (b + 2) % 4
                ng = g + (b + 2) // 4

                @pl.when(ng < NT)
                def _pf():
                    pltpu.make_async_copy(
                        x_hbm.at[pl.ds(x_off(ng, nb), T)],
                        xv[s], sin[s]).start()
                if b == 3:
                    npos = g + 2
                    @pl.when(npos < NT)
                    def _pp():
                        pltpu.make_async_copy(
                            pos_hbm.at[pl.ds(p_off(npos), T)],
                            pv[q], spos[q]).start()
        return carry

    lax.fori_loop(0, NT // 2, group2, 0)

    pltpu.make_async_copy(
        ov0, out_hbm.at[pl.ds(x_off(NT - 1, 2), T)], sout0).wait()
    pltpu.make_async_copy(
        ov1, out_hbm.at[pl.ds(x_off(NT - 1, 3), T)], sout1).wait()


def kernel(x, pos_table):
    xf = x.reshape(-1)
    pf = pos_table.reshape(-1)
    mesh = plsc.VectorSubcoreMesh(core_axis_name="c", subcore_axis_name="s")
    run = functools.partial(
        pl.kernel,
        mesh=mesh,
        out_type=jax.ShapeDtypeStruct((B * S * D,), jnp.float32),
        scratch_types=[
            pltpu.VMEM((T,), jnp.float32),
            pltpu.VMEM((T,), jnp.float32),
            pltpu.VMEM((T,), jnp.float32),
            pltpu.VMEM((T,), jnp.float32),
            pltpu.VMEM((T,), jnp.float32),
            pltpu.VMEM((T,), jnp.float32),
            pltpu.SemaphoreType.DMA,
            pltpu.SemaphoreType.DMA,
            pltpu.SemaphoreType.DMA,
            pltpu.SemaphoreType.DMA,
            pltpu.SemaphoreType.DMA,
            pltpu.SemaphoreType.DMA,
        ],
    )(_sc_body)
    return run(xf, pf).reshape(B, S, D)


# speedup vs baseline: 6.1659x; 4.2008x over previous
import jax
import jax.numpy as jnp
from jax.experimental import pallas as pl
from jax.experimental.pallas import tpu as pltpu


def _body(x_ref, p_ref, o_ref):
    o_ref[...] = x_ref[...] + p_ref[...]


def kernel(x, pos_table):
    b, s, d = x.shape
    sb = 2048
    grid = (s // sb, b)
    return pl.pallas_call(
        _body,
        grid=grid,
        in_specs=[
            pl.BlockSpec((1, sb, d), lambda i, j: (j, i, 0)),
            pl.BlockSpec((sb, d), lambda i, j: (i, 0)),
        ],
        out_specs=pl.BlockSpec((1, sb, d), lambda i, j: (j, i, 0)),
        out_shape=jax.ShapeDtypeStruct((b, s, d), x.dtype),
        compiler_params=pltpu.CompilerParams(
            dimension_semantics=("parallel", "parallel"),
        ),
    )(x, pos_table)
